# Initial kernel scaffold; baseline (speedup 1.0000x reference)
import jax
import jax.numpy as jnp
from jax.experimental import pallas as pl


def kernel(X, node_idx, edge_idx):
    raise NotImplementedError("write your pallas kernel here")


# speedup vs baseline: 7.0565x; 7.0565x over previous
import functools

import jax
import jax.numpy as jnp
from jax import lax
from jax.experimental import pallas as pl
from jax.experimental.pallas import tpu as pltpu
from jax.experimental.pallas import tpu_sc as plsc

N_V = 10000
N_E = 5000
NNZ = 320000
D = 128
ND8 = D // 16

NC, NS, L = 2, 16, 16
NW = NC * NS
CHUNK = NNZ // NW
BATCH = 128
NFULL = CHUNK // BATCH
TAIL = CHUNK - NFULL * BATCH

HV_PAD = 10240
HE_PAD = 5120
N_EP = 5120
N_VP = 10240

_mesh = plsc.VectorSubcoreMesh(core_axis_name="c", subcore_axis_name="s")


def _wid():
    return lax.axis_index("s") * NC + lax.axis_index("c")


def _zeros16():
    return jnp.zeros((L,), jnp.float32)


def _fill_zbuf(zbuf):
    z = _zeros16()
    for r in range(L):
        for t in range(ND8):
            zbuf[r, pl.ds(16 * t, 16)] = z


def _stage_batch(dst, src, base):
    for t in range(BATCH // 16):
        dst[pl.ds(16 * t, 16)] = src[pl.ds(base + 16 * t, 16)]


@functools.partial(
    pl.kernel,
    out_type=(
        jax.ShapeDtypeStruct((NC * HV_PAD,), jnp.float32),
        jax.ShapeDtypeStruct((NC * HE_PAD,), jnp.float32),
    ),
    mesh=_mesh,
    scratch_types=[
        pltpu.VMEM((CHUNK,), jnp.int32),
        pltpu.VMEM((CHUNK,), jnp.int32),
        pltpu.VMEM((BATCH,), jnp.int32),
        pltpu.VMEM((BATCH,), jnp.int32),
        pltpu.VMEM((16,), jnp.int32),
        pltpu.VMEM((16,), jnp.int32),
        pltpu.VMEM((BATCH,), jnp.float32),
        pltpu.VMEM((640,), jnp.float32),
        pltpu.VMEM_SHARED((HV_PAD,), jnp.float32),
        pltpu.VMEM_SHARED((HE_PAD,), jnp.float32),
    ],
)
def _degrees(nidx, eidx, degv_out, dege_out,
             nchunk, echunk, nbuf, ebuf, ntail, etail, ones, zb, hv, he):
    c = lax.axis_index("c")
    s = lax.axis_index("s")
    wid = _wid()

    one = jnp.ones((16,), jnp.float32)
    z = _zeros16()
    for t in range(BATCH // 16):
        ones[pl.ds(16 * t, 16)] = one
    for t in range(640 // 16):
        zb[pl.ds(16 * t, 16)] = z

    pltpu.sync_copy(zb, hv.at[pl.ds(640 * s, 640)])
    pltpu.sync_copy(zb.at[pl.ds(0, 320)], he.at[pl.ds(320 * s, 320)])
    plsc.subcore_barrier()

    base0 = wid * CHUNK
    pltpu.sync_copy(nidx.at[pl.ds(base0, CHUNK)], nchunk)
    pltpu.sync_copy(eidx.at[pl.ds(base0, CHUNK)], echunk)

    def body(j, carry):
        base = j * BATCH
        _stage_batch(nbuf, nchunk, base)
        _stage_batch(ebuf, echunk, base)
        pltpu.sync_copy(ones, hv.at[nbuf], add=True)
        pltpu.sync_copy(ones, he.at[ebuf], add=True)
        return carry

    lax.fori_loop(0, NFULL, body, 0)

    tb = NFULL * BATCH
    ntail[pl.ds(0, 16)] = nchunk[pl.ds(tb, 16)]
    etail[pl.ds(0, 16)] = echunk[pl.ds(tb, 16)]
    pltpu.sync_copy(ones.at[pl.ds(0, 16)], hv.at[ntail], add=True)
    pltpu.sync_copy(ones.at[pl.ds(0, 16)], he.at[etail], add=True)

    plsc.subcore_barrier()
    pltpu.sync_copy(hv.at[pl.ds(640 * s, 640)], zb)
    pltpu.sync_copy(zb, degv_out.at[pl.ds(c * HV_PAD + 640 * s, 640)])
    pltpu.sync_copy(he.at[pl.ds(320 * s, 320)], zb.at[pl.ds(0, 320)])
    pltpu.sync_copy(zb.at[pl.ds(0, 320)],
                    dege_out.at[pl.ds(c * HE_PAD + 320 * s, 320)])


@functools.partial(
    pl.kernel,
    out_type=jax.ShapeDtypeStruct((N_V, D), jnp.float32),
    mesh=_mesh,
    scratch_types=[
        pltpu.VMEM((16, D), jnp.float32),
        pltpu.VMEM((16,), jnp.float32),
        pltpu.VMEM((16,), jnp.float32),
    ],
)
def _normalize_x(x, degv_part, xnorm, rows, d0, d1):
    wid = _wid()
    ngroups = N_V // 16
    trip = (ngroups - wid + NW - 1) // NW

    def body(k, carry):
        g = wid + NW * k
        r0 = 16 * g
        pltpu.sync_copy(x.at[pl.ds(r0, 16)], rows)
        pltpu.sync_copy(degv_part.at[pl.ds(r0, 16)], d0)
        pltpu.sync_copy(degv_part.at[pl.ds(HV_PAD + r0, 16)], d1)
        dv = d0[pl.ds(0, 16)] + d1[pl.ds(0, 16)]
        inv = jnp.where(dv > 0, 1.0 / dv, 0.0)
        for r in range(16):
            sv = jnp.broadcast_to(inv[r], (16,))
            for t in range(ND8):
                rows[r, pl.ds(16 * t, 16)] = rows[r, pl.ds(16 * t, 16)] * sv
        pltpu.sync_copy(rows, xnorm.at[pl.ds(r0, 16)])
        return carry

    lax.fori_loop(0, trip, body, 0)


def _make_aggregate(n_out_pad):
    rows_per_tile = n_out_pad // NS
    nfull16 = rows_per_tile // 16
    assert rows_per_tile % 16 == 0

    @functools.partial(
        pl.kernel,
        out_type=jax.ShapeDtypeStruct((NC, n_out_pad, D), jnp.float32),
        mesh=_mesh,
        scratch_types=[
            pltpu.VMEM((CHUNK,), jnp.int32),
            pltpu.VMEM((CHUNK,), jnp.int32),
            pltpu.VMEM((BATCH,), jnp.int32),
            pltpu.VMEM((BATCH,), jnp.int32),
            pltpu.VMEM((16,), jnp.int32),
            pltpu.VMEM((16,), jnp.int32),
            pltpu.VMEM((BATCH, D), jnp.float32),
            pltpu.VMEM((16, D), jnp.float32),
            pltpu.VMEM_SHARED((n_out_pad, D), jnp.float32),
            pltpu.SemaphoreType.DMA,
        ],
    )
    def agg(table, gidx, sidx, part_out,
            gchunk, schunk, gbuf, sbuf, gtail, stail, rows, zb, acc, sem):
        c = lax.axis_index("c")
        s = lax.axis_index("s")
        wid = _wid()

        _fill_zbuf(zb)
        r0 = rows_per_tile * s
        for k in range(nfull16):
            pltpu.sync_copy(zb, acc.at[pl.ds(r0 + 16 * k, 16)])
        plsc.subcore_barrier()

        base0 = wid * CHUNK
        pltpu.sync_copy(gidx.at[pl.ds(base0, CHUNK)], gchunk)
        pltpu.sync_copy(sidx.at[pl.ds(base0, CHUNK)], schunk)

        def body(j, carry):
            base = j * BATCH
            _stage_batch(gbuf, gchunk, base)
            _stage_batch(sbuf, schunk, base)
            pltpu.async_copy(table.at[gbuf], rows, sem).wait()
            pltpu.sync_copy(rows, acc.at[sbuf], add=True)
            return carry

        lax.fori_loop(0, NFULL, body, 0)

        tb = NFULL * BATCH
        gtail[pl.ds(0, 16)] = gchunk[pl.ds(tb, 16)]
        stail[pl.ds(0, 16)] = schunk[pl.ds(tb, 16)]
        pltpu.async_copy(table.at[gtail], zb, sem).wait()
        pltpu.sync_copy(zb, acc.at[stail], add=True)

        plsc.subcore_barrier()
        st = rows.at[pl.ds(0, 64)]
        for k in range(rows_per_tile // 64):
            pltpu.sync_copy(acc.at[pl.ds(r0 + 64 * k, 64)], st)
            pltpu.sync_copy(st, part_out.at[c, pl.ds(r0 + 64 * k, 64)])

    return agg


_v2e = _make_aggregate(N_EP)
_e2v = _make_aggregate(N_VP)


@functools.partial(
    pl.kernel,
    out_type=(
        jax.ShapeDtypeStruct((N_E, D), jnp.float32),
        jax.ShapeDtypeStruct((N_EP, D), jnp.float32),
    ),
    mesh=_mesh,
    scratch_types=[
        pltpu.VMEM((16, D), jnp.float32),
        pltpu.VMEM((16, D), jnp.float32),
        pltpu.VMEM((16,), jnp.float32),
        pltpu.VMEM((16,), jnp.float32),
    ],
)
def _combine_edges(ef_part, dege_part, edge_feat, efn, ba, bb, d0, d1):
    wid = _wid()
    ngroups = 313
    trip = (ngroups - wid + NW - 1) // NW

    def body(k, carry):
        g = wid + NW * k
        r0 = 16 * g
        pltpu.sync_copy(ef_part.at[0, pl.ds(r0, 16)], ba)
        pltpu.sync_copy(ef_part.at[1, pl.ds(r0, 16)], bb)
        pltpu.sync_copy(dege_part.at[pl.ds(r0, 16)], d0)
        pltpu.sync_copy(dege_part.at[pl.ds(HE_PAD + r0, 16)], d1)
        dv = d0[pl.ds(0, 16)] + d1[pl.ds(0, 16)]
        inv = jnp.where(dv > 0, 1.0 / dv, 0.0)
        for r in range(16):
            for t in range(ND8):
                ba[r, pl.ds(16 * t, 16)] = (ba[r, pl.ds(16 * t, 16)]
                                            + bb[r, pl.ds(16 * t, 16)])

        @pl.when(g < ngroups - 1)
        def _full():
            pltpu.sync_copy(ba, edge_feat.at[pl.ds(r0, 16)])

        @pl.when(g == ngroups - 1)
        def _tail():
            pltpu.sync_copy(ba.at[pl.ds(0, 8)], edge_feat.at[pl.ds(r0, 8)])

        for r in range(16):
            sv = jnp.broadcast_to(inv[r], (16,))
            for t in range(ND8):
                bb[r, pl.ds(16 * t, 16)] = ba[r, pl.ds(16 * t, 16)] * sv
        pltpu.sync_copy(bb, efn.at[pl.ds(r0, 16)])
        return carry

    lax.fori_loop(0, trip, body, 0)


@functools.partial(
    pl.kernel,
    out_type=jax.ShapeDtypeStruct((N_V, D), jnp.float32),
    mesh=_mesh,
    scratch_types=[
        pltpu.VMEM((16, D), jnp.float32),
        pltpu.VMEM((16, D), jnp.float32),
    ],
)
def _combine_nodes(nf_part, node_feat, ba, bb):
    wid = _wid()
    ngroups = N_V // 16
    trip = (ngroups - wid + NW - 1) // NW

    def body(k, carry):
        g = wid + NW * k
        r0 = 16 * g
        pltpu.sync_copy(nf_part.at[0, pl.ds(r0, 16)], ba)
        pltpu.sync_copy(nf_part.at[1, pl.ds(r0, 16)], bb)
        for r in range(16):
            for t in range(ND8):
                ba[r, pl.ds(16 * t, 16)] = (ba[r, pl.ds(16 * t, 16)]
                                            + bb[r, pl.ds(16 * t, 16)])
        pltpu.sync_copy(ba, node_feat.at[pl.ds(r0, 16)])
        return carry

    lax.fori_loop(0, trip, body, 0)


def kernel(X, node_idx, edge_idx):
    node_idx = node_idx.astype(jnp.int32)
    edge_idx = edge_idx.astype(jnp.int32)
    degv_part, dege_part = _degrees(node_idx, edge_idx)
    xnorm = _normalize_x(X, degv_part)
    ef_part = _v2e(xnorm, node_idx, edge_idx)
    edge_feat, efn = _combine_edges(ef_part, dege_part)
    nf_part = _e2v(efn, edge_idx, node_idx)
    node_feat = _combine_nodes(nf_part)
    return (node_feat, edge_feat)
